# baseline (speedup 1.0000x reference)
import jax
import jax.numpy as jnp
import numpy as np
from jax.experimental import pallas as pl

_SIGMA = 4.0
_R = int(np.ceil(3.0 * _SIGMA))
_N = 512


def _blur_matrix() -> jnp.ndarray:
    x = np.arange(-_R, _R + 1, dtype=np.float64)
    w = np.exp(-0.5 * (x / _SIGMA) ** 2)
    w = w / w.sum()
    b = np.zeros((_N, _N), dtype=np.float64)
    rows = np.arange(_N)
    for t in range(2 * _R + 1):
        cols = np.clip(rows + t - _R, 0, _N - 1)
        np.add.at(b, (rows, cols), w[t])
    return b.astype(np.float32)


_B = _blur_matrix()


def _blur_body(img_ref, b_ref, out_ref):
    img = img_ref[0]
    b = b_ref[...]
    tmp = jax.lax.dot(b, img, precision=jax.lax.Precision.DEFAULT,
                      preferred_element_type=jnp.float32)
    out = jax.lax.dot_general(
        tmp, b, (((1,), (1,)), ((), ())),
        precision=jax.lax.Precision.DEFAULT,
        preferred_element_type=jnp.float32)
    out_ref[0] = out


@jax.jit
def kernel(img):
    return pl.pallas_call(
        _blur_body,
        out_shape=jax.ShapeDtypeStruct((1, _N, _N), jnp.float32),
    )(img, jnp.asarray(_B))


# speedup vs baseline: 1.0627x; 1.0627x over previous
import jax
import jax.numpy as jnp
import numpy as np
from jax.experimental import pallas as pl

_SIGMA = 4.0
_R = int(np.ceil(3.0 * _SIGMA))
_N = 512


def _blur_matrix() -> jnp.ndarray:
    x = np.arange(-_R, _R + 1, dtype=np.float64)
    w = np.exp(-0.5 * (x / _SIGMA) ** 2)
    w = w / w.sum()
    b = np.zeros((_N, _N), dtype=np.float64)
    rows = np.arange(_N)
    for t in range(2 * _R + 1):
        cols = np.clip(rows + t - _R, 0, _N - 1)
        np.add.at(b, (rows, cols), w[t])
    return b


_B = _blur_matrix()


def _blur_body(img_ref, b_ref, out_ref):
    img = img_ref[0]
    b = b_ref[...].astype(jnp.float32)
    tmp = jax.lax.dot(b, img, precision=jax.lax.Precision.DEFAULT,
                      preferred_element_type=jnp.float32)
    out = jax.lax.dot_general(
        tmp, b, (((1,), (1,)), ((), ())),
        precision=jax.lax.Precision.DEFAULT,
        preferred_element_type=jnp.float32)
    out_ref[0] = out


@jax.jit
def kernel(img):
    return pl.pallas_call(
        _blur_body,
        out_shape=jax.ShapeDtypeStruct((1, _N, _N), jnp.float32),
    )(img, jnp.asarray(_B, dtype=jnp.bfloat16))
